# Initial kernel scaffold; baseline (speedup 1.0000x reference)
import jax
import jax.numpy as jnp
from jax.experimental import pallas as pl


def kernel(node_embeddings, atoms_adj, atoms_goal_adj, atoms_label, Wi_adj, bi_adj, Wo_adj, bo_adj, Wi_goal_adj, bi_goal_adj, Wo_goal_adj, bo_goal_adj, Wi_label, bi_label, Wo_label, bo_label, Wi_upd, bi_upd, Wo_upd, bo_upd):
    raise NotImplementedError("write your pallas kernel here")


# speedup vs baseline: 2.0201x; 2.0201x over previous
import functools

import jax
import jax.numpy as jnp
from jax import lax
from jax.experimental import pallas as pl
from jax.experimental.pallas import tpu as pltpu
from jax.experimental.pallas import tpu_sc as plsc

N = 10000
D = 128
E_PAIR = 300000
E_LAB = 10000
B = 2 * E_PAIR + E_LAB
CHUNK = 128
NW = 32
CPW = 150
B_PAD = NW * CPW * CHUNK
CHUNKS_TOTAL = B_PAD // CHUNK
ACC_ROWS = 10112
RPT = ACC_ROWS // 16
BLK = 200
NBLK = (B_PAD // 2) // BLK
UBLK = 1000

_SC_MESH = plsc.VectorSubcoreMesh(core_axis_name="c", subcore_axis_name="s")


def _worker_id():
    return lax.axis_index("s") * 2 + lax.axis_index("c")


@functools.partial(
    pl.kernel,
    out_type=jax.ShapeDtypeStruct((B_PAD, D), jnp.float32),
    mesh=_SC_MESH,
    scratch_types=[
        pltpu.VMEM((CPW, CHUNK), jnp.int32),
        pltpu.VMEM((CHUNK, D), jnp.float32),
        pltpu.SemaphoreType.DMA,
    ],
)
def _gather_k(tbl_hbm, idx_hbm, out_hbm, idx_v, buf, sem):
    w = _worker_id()
    c0 = w * CPW
    pltpu.sync_copy(idx_hbm.at[w], idx_v)

    def body(t, carry):
        pltpu.async_copy(tbl_hbm.at[idx_v.at[t]], buf, sem).wait()
        pltpu.sync_copy(buf, out_hbm.at[pl.ds((c0 + t) * CHUNK, CHUNK)])
        return carry

    lax.fori_loop(0, CPW, body, 0)


@functools.partial(
    pl.kernel,
    out_type=jax.ShapeDtypeStruct((2, ACC_ROWS, D), jnp.float32),
    mesh=_SC_MESH,
    scratch_types=[
        pltpu.VMEM((CPW, CHUNK), jnp.int32),
        pltpu.VMEM((CHUNK, D), jnp.float32),
        pltpu.VMEM_SHARED((ACC_ROWS, D), jnp.float32),
        pltpu.SemaphoreType.DMA,
    ],
)
def _scatter_k(y_hbm, idx_hbm, zeros_hbm, out_hbm, idx_v, buf, acc, sem):
    c = lax.axis_index("c")
    s = lax.axis_index("s")
    w = s * 2 + c
    pltpu.sync_copy(zeros_hbm.at[pl.ds(s * RPT, RPT)], acc.at[pl.ds(s * RPT, RPT)])
    plsc.subcore_barrier()
    c0 = w * CPW
    pltpu.sync_copy(idx_hbm.at[w], idx_v)

    def body(t, carry):
        pltpu.sync_copy(y_hbm.at[pl.ds((c0 + t) * CHUNK, CHUNK)], buf)
        pltpu.sync_copy(buf, acc.at[idx_v.at[t]], add=True)
        return carry

    lax.fori_loop(0, CPW, body, 0)
    plsc.subcore_barrier()
    pltpu.sync_copy(acc.at[pl.ds(s * RPT, RPT)], out_hbm.at[c, pl.ds(s * RPT, RPT)])


def _mish(z):
    sp = jnp.maximum(z, 0.0) + jnp.log1p(jnp.exp(-jnp.abs(z)))
    return z * jnp.tanh(sp)


def _dot(a, b):
    return jnp.dot(a, b, preferred_element_type=jnp.float32,
                   precision=lax.Precision.HIGHEST)


def _mlp_body(x_ref, wi_ref, bi_ref, wo_ref, bo_ref, y_ref):
    x = x_ref[...]
    z = _mish(_dot(x, wi_ref[0]) + bi_ref[0])
    y_ref[...] = x + _dot(z, wo_ref[0]) + bo_ref[0]


def _rel_mlp(x2, wi_s, bi_s, wo_s, bo_s):
    wsel = lambda i: (jnp.minimum(i // 750, 2), 0, 0)
    return pl.pallas_call(
        _mlp_body,
        grid=(NBLK,),
        in_specs=[
            pl.BlockSpec((BLK, 2 * D), lambda i: (i, 0)),
            pl.BlockSpec((1, 2 * D, 2 * D), wsel),
            pl.BlockSpec((1, 1, 2 * D), wsel),
            pl.BlockSpec((1, 2 * D, 2 * D), wsel),
            pl.BlockSpec((1, 1, 2 * D), wsel),
        ],
        out_specs=pl.BlockSpec((BLK, 2 * D), lambda i: (i, 0)),
        out_shape=jax.ShapeDtypeStruct((B_PAD // 2, 2 * D), jnp.float32),
    )(x2, wi_s, bi_s, wo_s, bo_s)


def _upd_body(p0_ref, p1_ref, h_ref, wt_ref, wb_ref, bi_ref, wo_ref, bo_ref, o_ref):
    sm = p0_ref[0] + p1_ref[0]
    h = h_ref[...]
    z = _mish(_dot(sm, wt_ref[...]) + _dot(h, wb_ref[...]) + bi_ref[...])
    o_ref[...] = h + _dot(z, wo_ref[...]) + bo_ref[...]


def _update(partials, h, wt, wb, bi, wo, bo):
    return pl.pallas_call(
        _upd_body,
        grid=(N // UBLK,),
        in_specs=[
            pl.BlockSpec((1, UBLK, D), lambda i: (0, i, 0)),
            pl.BlockSpec((1, UBLK, D), lambda i: (1, i, 0)),
            pl.BlockSpec((UBLK, D), lambda i: (i, 0)),
            pl.BlockSpec((D, 2 * D), lambda i: (0, 0)),
            pl.BlockSpec((D, 2 * D), lambda i: (0, 0)),
            pl.BlockSpec((1, 2 * D), lambda i: (0, 0)),
            pl.BlockSpec((2 * D, D), lambda i: (0, 0)),
            pl.BlockSpec((1, D), lambda i: (0, 0)),
        ],
        out_specs=pl.BlockSpec((UBLK, D), lambda i: (i, 0)),
        out_shape=jax.ShapeDtypeStruct((N, D), jnp.float32),
    )(partials, partials, h, wt, wb, bi, wo, bo)


def _blockdiag(w):
    z = jnp.zeros((2 * D, 2 * D), jnp.float32)
    return z.at[:D, :D].set(w).at[D:, D:].set(w)


def kernel(node_embeddings, atoms_adj, atoms_goal_adj, atoms_label,
           Wi_adj, bi_adj, Wo_adj, bo_adj,
           Wi_goal_adj, bi_goal_adj, Wo_goal_adj, bo_goal_adj,
           Wi_label, bi_label, Wo_label, bo_label,
           Wi_upd, bi_upd, Wo_upd, bo_upd):
    idx = jnp.concatenate([atoms_adj, atoms_goal_adj, atoms_label]).astype(jnp.int32)
    gidx = jnp.concatenate(
        [idx, jnp.zeros((B_PAD - B,), jnp.int32)]).reshape(NW, CPW, CHUNK)
    sidx = jnp.concatenate(
        [idx, jnp.full((B_PAD - B,), N, jnp.int32)]).reshape(NW, CPW, CHUNK)
    zeros_acc = jnp.zeros((ACC_ROWS, D), jnp.float32)

    wi_s = jnp.stack([Wi_adj, Wi_goal_adj, _blockdiag(Wi_label)])
    wo_s = jnp.stack([Wo_adj, Wo_goal_adj, _blockdiag(Wo_label)])
    bi_s = jnp.stack([bi_adj, bi_goal_adj,
                      jnp.concatenate([bi_label, bi_label])]).reshape(3, 1, 2 * D)
    bo_s = jnp.stack([bo_adj, bo_goal_adj,
                      jnp.concatenate([bo_label, bo_label])]).reshape(3, 1, 2 * D)

    wt = Wi_upd[:D]
    wb = Wi_upd[D:]
    bi_u = bi_upd.reshape(1, 2 * D)
    bo_u = bo_upd.reshape(1, D)

    h = node_embeddings
    for _ in range(2):
        x = _gather_k(h, gidx)
        y2 = _rel_mlp(x.reshape(B_PAD // 2, 2 * D), wi_s, bi_s, wo_s, bo_s)
        partials = _scatter_k(y2.reshape(B_PAD, D), sidx, zeros_acc)
        h = _update(partials, h, wt, wb, bi_u, Wo_upd, bo_u)
    return h


# speedup vs baseline: 2.4853x; 1.2303x over previous
import functools

import jax
import jax.numpy as jnp
from jax import lax
from jax.experimental import pallas as pl
from jax.experimental.pallas import tpu as pltpu
from jax.experimental.pallas import tpu_sc as plsc

N = 10000
D = 128
E_PAIR = 300000
E_LAB = 10000
B = 2 * E_PAIR + E_LAB
CHUNK = 128
NW = 32
CPW = 150
B_PAD = NW * CPW * CHUNK
CHUNKS_TOTAL = B_PAD // CHUNK
ACC_ROWS = 10112
RPT = ACC_ROWS // 16
BLK = 200
NBLK = (B_PAD // 2) // BLK
UBLK = 1000

_SC_MESH = plsc.VectorSubcoreMesh(core_axis_name="c", subcore_axis_name="s")


def _worker_id():
    return lax.axis_index("s") * 2 + lax.axis_index("c")


@functools.partial(
    pl.kernel,
    out_type=jax.ShapeDtypeStruct((B_PAD, D), jnp.float32),
    mesh=_SC_MESH,
    scratch_types=[
        pltpu.VMEM((CPW, CHUNK), jnp.int32),
        pltpu.VMEM((CHUNK, D), jnp.float32),
        pltpu.SemaphoreType.DMA,
    ],
)
def _gather_k(tbl_hbm, idx_hbm, out_hbm, idx_v, buf, sem):
    w = _worker_id()
    c0 = w * CPW
    pltpu.sync_copy(idx_hbm.at[w], idx_v)

    def body(t, carry):
        pltpu.async_copy(tbl_hbm.at[idx_v.at[t]], buf, sem).wait()
        pltpu.sync_copy(buf, out_hbm.at[pl.ds((c0 + t) * CHUNK, CHUNK)])
        return carry

    lax.fori_loop(0, CPW, body, 0)


@functools.partial(
    pl.kernel,
    out_type=jax.ShapeDtypeStruct((2, ACC_ROWS, D), jnp.float32),
    mesh=_SC_MESH,
    scratch_types=[
        pltpu.VMEM((CPW, CHUNK), jnp.int32),
        pltpu.VMEM((CHUNK, D), jnp.float32),
        pltpu.VMEM_SHARED((ACC_ROWS, D), jnp.float32),
        pltpu.SemaphoreType.DMA,
    ],
)
def _scatter_k(y_hbm, idx_hbm, zeros_hbm, out_hbm, idx_v, buf, acc, sem):
    c = lax.axis_index("c")
    s = lax.axis_index("s")
    w = s * 2 + c
    pltpu.sync_copy(zeros_hbm.at[pl.ds(s * RPT, RPT)], acc.at[pl.ds(s * RPT, RPT)])
    plsc.subcore_barrier()
    c0 = w * CPW
    pltpu.sync_copy(idx_hbm.at[w], idx_v)

    def body(t, carry):
        pltpu.sync_copy(y_hbm.at[pl.ds((c0 + t) * CHUNK, CHUNK)], buf)
        pltpu.sync_copy(buf, acc.at[idx_v.at[t]], add=True)
        return carry

    lax.fori_loop(0, CPW, body, 0)
    plsc.subcore_barrier()
    pltpu.sync_copy(acc.at[pl.ds(s * RPT, RPT)], out_hbm.at[c, pl.ds(s * RPT, RPT)])


def _mish(z):
    sp = jnp.maximum(z, 0.0) + jnp.log1p(jnp.exp(-jnp.abs(z)))
    return z * jnp.tanh(sp)


def _dot(a, b):
    return jnp.dot(a, b, preferred_element_type=jnp.float32,
                   precision=lax.Precision.DEFAULT)


def _mlp_body(x_ref, wi_ref, bi_ref, wo_ref, bo_ref, y_ref):
    x = x_ref[...]
    z = _mish(_dot(x, wi_ref[0]) + bi_ref[0])
    y_ref[...] = x + _dot(z, wo_ref[0]) + bo_ref[0]


def _rel_mlp(x2, wi_s, bi_s, wo_s, bo_s):
    wsel = lambda i: (jnp.minimum(i // 750, 2), 0, 0)
    return pl.pallas_call(
        _mlp_body,
        grid=(NBLK,),
        in_specs=[
            pl.BlockSpec((BLK, 2 * D), lambda i: (i, 0)),
            pl.BlockSpec((1, 2 * D, 2 * D), wsel),
            pl.BlockSpec((1, 1, 2 * D), wsel),
            pl.BlockSpec((1, 2 * D, 2 * D), wsel),
            pl.BlockSpec((1, 1, 2 * D), wsel),
        ],
        out_specs=pl.BlockSpec((BLK, 2 * D), lambda i: (i, 0)),
        out_shape=jax.ShapeDtypeStruct((B_PAD // 2, 2 * D), jnp.float32),
    )(x2, wi_s, bi_s, wo_s, bo_s)


def _upd_body(p0_ref, p1_ref, h_ref, wt_ref, wb_ref, bi_ref, wo_ref, bo_ref, o_ref):
    sm = p0_ref[0] + p1_ref[0]
    h = h_ref[...]
    z = _mish(_dot(sm, wt_ref[...]) + _dot(h, wb_ref[...]) + bi_ref[...])
    o_ref[...] = h + _dot(z, wo_ref[...]) + bo_ref[...]


def _update(partials, h, wt, wb, bi, wo, bo):
    return pl.pallas_call(
        _upd_body,
        grid=(N // UBLK,),
        in_specs=[
            pl.BlockSpec((1, UBLK, D), lambda i: (0, i, 0)),
            pl.BlockSpec((1, UBLK, D), lambda i: (1, i, 0)),
            pl.BlockSpec((UBLK, D), lambda i: (i, 0)),
            pl.BlockSpec((D, 2 * D), lambda i: (0, 0)),
            pl.BlockSpec((D, 2 * D), lambda i: (0, 0)),
            pl.BlockSpec((1, 2 * D), lambda i: (0, 0)),
            pl.BlockSpec((2 * D, D), lambda i: (0, 0)),
            pl.BlockSpec((1, D), lambda i: (0, 0)),
        ],
        out_specs=pl.BlockSpec((UBLK, D), lambda i: (i, 0)),
        out_shape=jax.ShapeDtypeStruct((N, D), jnp.float32),
    )(partials, partials, h, wt, wb, bi, wo, bo)


def _blockdiag(w):
    z = jnp.zeros((2 * D, 2 * D), jnp.float32)
    return z.at[:D, :D].set(w).at[D:, D:].set(w)


def kernel(node_embeddings, atoms_adj, atoms_goal_adj, atoms_label,
           Wi_adj, bi_adj, Wo_adj, bo_adj,
           Wi_goal_adj, bi_goal_adj, Wo_goal_adj, bo_goal_adj,
           Wi_label, bi_label, Wo_label, bo_label,
           Wi_upd, bi_upd, Wo_upd, bo_upd):
    idx = jnp.concatenate([atoms_adj, atoms_goal_adj, atoms_label]).astype(jnp.int32)
    gidx = jnp.concatenate(
        [idx, jnp.zeros((B_PAD - B,), jnp.int32)]).reshape(NW, CPW, CHUNK)
    sidx = jnp.concatenate(
        [idx, jnp.full((B_PAD - B,), N, jnp.int32)]).reshape(NW, CPW, CHUNK)
    zeros_acc = jnp.zeros((ACC_ROWS, D), jnp.float32)

    wi_s = jnp.stack([Wi_adj, Wi_goal_adj, _blockdiag(Wi_label)])
    wo_s = jnp.stack([Wo_adj, Wo_goal_adj, _blockdiag(Wo_label)])
    bi_s = jnp.stack([bi_adj, bi_goal_adj,
                      jnp.concatenate([bi_label, bi_label])]).reshape(3, 1, 2 * D)
    bo_s = jnp.stack([bo_adj, bo_goal_adj,
                      jnp.concatenate([bo_label, bo_label])]).reshape(3, 1, 2 * D)

    wt = Wi_upd[:D]
    wb = Wi_upd[D:]
    bi_u = bi_upd.reshape(1, 2 * D)
    bo_u = bo_upd.reshape(1, D)

    h = node_embeddings
    for _ in range(2):
        x = _gather_k(h, gidx)
        y2 = _rel_mlp(x.reshape(B_PAD // 2, 2 * D), wi_s, bi_s, wo_s, bo_s)
        partials = _scatter_k(y2.reshape(B_PAD, D), sidx, zeros_acc)
        h = _update(partials, h, wt, wb, bi_u, Wo_upd, bo_u)
    return h
